# baseline (speedup 1.0000x reference)
import functools
import math

import jax
import jax.numpy as jnp
from jax import lax
from jax.experimental import pallas as pl
from jax.experimental.pallas import tpu as pltpu
from jax.experimental.pallas import tpu_sc as plsc

D_MODEL = 64
BPT = 128
NBUF = 4
LOOKG = 2
SCALE = math.sqrt(D_MODEL)


@functools.cache
def _build(batch: int, hist: int):
    info = plsc.get_sparse_core_info()
    nc, ns = info.num_cores, info.num_subcores
    nw = nc * ns
    assert batch == nw * BPT
    steps = hist

    mesh = plsc.VectorSubcoreMesh(core_axis_name="c", subcore_axis_name="s")

    @functools.partial(
        pl.kernel,
        mesh=mesh,
        out_type=jax.ShapeDtypeStruct(
            (hist, D_MODEL // 8, batch // 128, 8, 128), jnp.float32),
        compiler_params=pltpu.CompilerParams(
            use_tc_tiling_on_sc=False, needs_layout_passes=False),
        scratch_types=[
            pltpu.VMEM((hist, BPT), jnp.int32),
            [pltpu.VMEM((BPT, D_MODEL), jnp.float32) for _ in range(NBUF)],
            [pltpu.VMEM((D_MODEL // 8, 8, BPT), jnp.float32)
             for _ in range(NBUF)],
            [pltpu.SemaphoreType.DMA for _ in range(NBUF)],
            [pltpu.SemaphoreType.DMA for _ in range(NBUF)],
        ],
    )
    def emb(x_hbm, table_hbm, out_hbm, idx_v, gbufs, obufs, gsems, ssems):
        wid = lax.axis_index("s") * nc + lax.axis_index("c")
        pltpu.sync_copy(x_hbm.at[wid], idx_v)

        def gather(h, slot):
            pltpu.async_copy(table_hbm.at[idx_v.at[h]], gbufs[slot],
                             gsems[slot])

        for s in range(LOOKG):
            gather(s, s % NBUF)

        iota16 = lax.iota(jnp.int32, 16)
        rowvecs = [iota16 + (16 * grp) for grp in range(BPT // 16)]

        def outer(o4, carry):
            for u in range(NBUF):
                s = o4 * NBUF + u

                @pl.when(s >= NBUF)
                def _():
                    pltpu.make_async_copy(
                        obufs[u], out_hbm.at[0, :, 0], ssems[u]).wait()

                pltpu.make_async_copy(
                    table_hbm.at[idx_v.at[s]], gbufs[u], gsems[u]).wait()

                def d_body(d, c2):
                    colvec = jnp.full((16,), 0, jnp.int32) + d
                    tr = d // 8
                    r = d % 8
                    for grp in range(BPT // 16):
                        v = plsc.load_gather(gbufs[u], [rowvecs[grp], colvec])
                        obufs[u][tr, r, pl.ds(grp * 16, 16)] = v * SCALE
                    return c2

                lax.fori_loop(0, D_MODEL, d_body, 0, unroll=2)

                pltpu.async_copy(obufs[u], out_hbm.at[s, :, wid], ssems[u])

                @pl.when(s + LOOKG < steps)
                def _():
                    gather(s + LOOKG, (u + LOOKG) % NBUF)
            return carry

        lax.fori_loop(0, steps // NBUF, outer, 0)

        for u in range(NBUF):
            pltpu.make_async_copy(
                obufs[u], out_hbm.at[0, :, 0], ssems[u]).wait()

    return emb


def kernel(x, table):
    b, h = x.shape
    nw = b // BPT
    xtt = x.T.reshape(h, nw, BPT).transpose(1, 0, 2)
    out5d = _build(b, h)(xtt, table)
    return out5d.transpose(2, 4, 0, 1, 3).reshape(b, h, D_MODEL)


# speedup vs baseline: 1.7112x; 1.7112x over previous
import functools
import math

import jax
import jax.numpy as jnp
from jax import lax
from jax.experimental import pallas as pl
from jax.experimental.pallas import tpu as pltpu
from jax.experimental.pallas import tpu_sc as plsc

D_MODEL = 64
BPT = 128
NBUF = 4
LOOKG = 2
SCALE = math.sqrt(D_MODEL)


@functools.cache
def _build(batch: int, hist: int):
    info = plsc.get_sparse_core_info()
    nc, ns = info.num_cores, info.num_subcores
    nw = nc * ns
    assert batch == nw * BPT
    steps = hist

    mesh = plsc.VectorSubcoreMesh(core_axis_name="c", subcore_axis_name="s")

    @functools.partial(
        pl.kernel,
        mesh=mesh,
        out_type=jax.ShapeDtypeStruct(
            (hist, D_MODEL // 8, batch // 128, 8, 128), jnp.float32),
        compiler_params=pltpu.CompilerParams(
            use_tc_tiling_on_sc=False, needs_layout_passes=False),
        scratch_types=[
            pltpu.VMEM((hist, BPT), jnp.int32),
            [pltpu.VMEM((BPT, D_MODEL), jnp.float32) for _ in range(NBUF)],
            [pltpu.VMEM((D_MODEL // 8, 8, BPT), jnp.float32)
             for _ in range(NBUF)],
            [pltpu.SemaphoreType.DMA for _ in range(NBUF)],
            [pltpu.SemaphoreType.DMA for _ in range(NBUF)],
        ],
    )
    def emb(x_hbm, table_hbm, out_hbm, idx_v, gbufs, obufs, gsems, ssems):
        wid = lax.axis_index("s") * nc + lax.axis_index("c")
        pltpu.sync_copy(x_hbm.at[wid], idx_v)

        def gather(h, slot):
            pltpu.async_copy(table_hbm.at[idx_v.at[h]], gbufs[slot],
                             gsems[slot])

        for s in range(LOOKG):
            gather(s, s % NBUF)

        iota16 = lax.iota(jnp.int32, 16)
        rowvecs = [iota16 + (16 * grp) for grp in range(BPT // 16)]

        def outer(o4, carry):
            for u in range(NBUF):
                s = o4 * NBUF + u

                @pl.when(s >= NBUF)
                def _():
                    pltpu.make_async_copy(
                        obufs[u], out_hbm.at[0, :, 0], ssems[u]).wait()

                pltpu.make_async_copy(
                    table_hbm.at[idx_v.at[s]], gbufs[u], gsems[u]).wait()

                def d_body(d0, c2):
                    colvec = (d0 + iota16) & (D_MODEL - 1)
                    trvec = lax.shift_right_logical(colvec, 3)
                    rvec = colvec & 7
                    for grp in range(BPT // 16):
                        v = plsc.load_gather(gbufs[u], [rowvecs[grp], colvec])
                        plsc.store_scatter(
                            obufs[u], [trvec, rvec, rowvecs[grp]], v * SCALE)
                    return c2

                lax.fori_loop(0, D_MODEL, d_body, 0, unroll=2)

                pltpu.async_copy(obufs[u], out_hbm.at[s, :, wid], ssems[u])

                @pl.when(s + LOOKG < steps)
                def _():
                    gather(s + LOOKG, (u + LOOKG) % NBUF)
            return carry

        lax.fori_loop(0, steps // NBUF, outer, 0)

        for u in range(NBUF):
            pltpu.make_async_copy(
                obufs[u], out_hbm.at[0, :, 0], ssems[u]).wait()

    return emb


def kernel(x, table):
    b, h = x.shape
    nw = b // BPT
    xtt = x.T.reshape(h, nw, BPT).transpose(1, 0, 2)
    out5d = _build(b, h)(xtt, table)
    return out5d.transpose(2, 4, 0, 1, 3).reshape(b, h, D_MODEL)


# speedup vs baseline: 1.8139x; 1.0600x over previous
import functools
import math

import jax
import jax.numpy as jnp
from jax import lax
from jax.experimental import pallas as pl
from jax.experimental.pallas import tpu as pltpu
from jax.experimental.pallas import tpu_sc as plsc

D_MODEL = 64
BPT = 128
NBUF = 4
LOOKG = 2
SCALE = math.sqrt(D_MODEL)


@functools.cache
def _build(batch: int, hist: int):
    info = plsc.get_sparse_core_info()
    nc, ns = info.num_cores, info.num_subcores
    nw = nc * ns
    assert batch == nw * BPT
    steps = hist

    mesh = plsc.VectorSubcoreMesh(core_axis_name="c", subcore_axis_name="s")

    @functools.partial(
        pl.kernel,
        mesh=mesh,
        out_type=jax.ShapeDtypeStruct(
            (hist, D_MODEL // 8, batch // 128, 8, 128), jnp.float32),
        compiler_params=pltpu.CompilerParams(
            use_tc_tiling_on_sc=False, needs_layout_passes=False),
        scratch_types=[
            pltpu.VMEM((hist, BPT), jnp.int32),
            [pltpu.VMEM((BPT, 2 * D_MODEL), jnp.float32) for _ in range(NBUF)],
            [pltpu.VMEM((D_MODEL // 8, 8, BPT), jnp.float32)
             for _ in range(NBUF)],
            [pltpu.SemaphoreType.DMA for _ in range(NBUF)],
            [pltpu.SemaphoreType.DMA for _ in range(NBUF)],
        ],
    )
    def emb(x_hbm, table_hbm, out_hbm, idx_v, gbufs, obufs, gsems, ssems):
        wid = lax.axis_index("s") * nc + lax.axis_index("c")
        pltpu.sync_copy(x_hbm.at[wid], idx_v)

        def gather(h, slot):
            pltpu.async_copy(table_hbm.at[idx_v.at[h]], gbufs[slot],
                             gsems[slot])

        for s in range(LOOKG):
            gather(s, s % NBUF)

        iota16 = lax.iota(jnp.int32, 16)
        rowvecs = [iota16 + (16 * grp) for grp in range(BPT // 16)]

        def outer(o4, carry):
            for u in range(NBUF):
                s = o4 * NBUF + u

                @pl.when(s >= NBUF)
                def _():
                    pltpu.make_async_copy(
                        obufs[u], out_hbm.at[0, :, 0], ssems[u]).wait()

                pltpu.make_async_copy(
                    table_hbm.at[idx_v.at[s]], gbufs[u], gsems[u]).wait()

                def d_body(d0, c2):
                    colvec = (d0 + iota16) & (D_MODEL - 1)
                    trvec = lax.shift_right_logical(colvec, 3)
                    rvec = colvec & 7
                    for grp in range(BPT // 16):
                        v = plsc.load_gather(gbufs[u], [rowvecs[grp], colvec])
                        plsc.store_scatter(
                            obufs[u], [trvec, rvec, rowvecs[grp]], v * SCALE)
                    return c2

                lax.fori_loop(0, D_MODEL, d_body, 0, unroll=2)

                pltpu.async_copy(obufs[u], out_hbm.at[s, :, wid], ssems[u])

                @pl.when(s + LOOKG < steps)
                def _():
                    gather(s + LOOKG, (u + LOOKG) % NBUF)
            return carry

        lax.fori_loop(0, steps // NBUF, outer, 0)

        for u in range(NBUF):
            pltpu.make_async_copy(
                obufs[u], out_hbm.at[0, :, 0], ssems[u]).wait()

    return emb


def kernel(x, table):
    b, h = x.shape
    nw = b // BPT
    xtt = x.T.reshape(h, nw, BPT).transpose(1, 0, 2)
    tpad = jnp.pad(table, ((0, 0), (0, D_MODEL)))
    out5d = _build(b, h)(xtt, tpad)
    return out5d.transpose(2, 4, 0, 1, 3).reshape(b, h, D_MODEL)
